# Initial kernel scaffold; baseline (speedup 1.0000x reference)
import jax
import jax.numpy as jnp
from jax.experimental import pallas as pl


def kernel(word_feat, sent_raw, edge_src, edge_dst, W_proj, Wq_ws, Wk_ws, F1_s, F2_s, Wq_sw, Wk_sw, F1_w, F2_w, W_head, b_head):
    raise NotImplementedError("write your pallas kernel here")


# speedup vs baseline: 7.0322x; 7.0322x over previous
import functools

import jax
import jax.numpy as jnp
from jax.experimental import pallas as pl
from jax.experimental.pallas import tpu as pltpu

_N_ROUNDS = 3
_PREC = jax.lax.Precision.HIGHEST


def _dot(a, b, tb=False):
    dn = (((1,), (1 if tb else 0,)), ((), ()))
    return jax.lax.dot_general(a, b, dn, precision=_PREC,
                               preferred_element_type=jnp.float32)


def _pick_bd(n):
    for bd in (256, 200, 128, 100, 64, 50, 40, 32, 16, 8):
        if n % bd == 0:
            return bd
    return n


def _matmul_body(x_ref, w_ref, o_ref):
    o_ref[...] = _dot(x_ref[...], w_ref[...])


def _matmul(x, w):
    n, kd = x.shape
    m = w.shape[1]
    bd = _pick_bd(n)
    return pl.pallas_call(
        _matmul_body,
        grid=(n // bd,),
        in_specs=[
            pl.BlockSpec((bd, kd), lambda i: (i, 0)),
            pl.BlockSpec((kd, m), lambda i: (0, 0)),
        ],
        out_specs=pl.BlockSpec((bd, m), lambda i: (i, 0)),
        out_shape=jax.ShapeDtypeStruct((n, m), jnp.float32),
    )(x, w)


def _gat_body(dst_ref, k_ref, c_ref, wq_ref, f1_ref, f2_ref, o_ref):
    dst = dst_ref[...]
    k = k_ref[...]
    c = c_ref[...]
    q = _dot(dst, wq_ref[...])
    t = _dot(q, k, tb=True)
    t = jnp.where(t >= 0, t, 0.2 * t)
    edge = c > 0
    neg = jnp.float32(-1e30)
    m = jnp.max(jnp.where(edge, t, neg), axis=1, keepdims=True)
    p = c * jnp.exp(jnp.where(edge, t - m, neg))
    den = jnp.sum(p, axis=1, keepdims=True)
    agg = _dot(p, k) / (den + 1e-9)
    h = jnp.where(agg > 0, agg, jnp.exp(jnp.minimum(agg, 0.0)) - 1.0) + dst
    h = h + _dot(jnp.maximum(_dot(h, f1_ref[...]), 0.0), f2_ref[...])
    mu = jnp.mean(h, axis=1, keepdims=True)
    var = jnp.mean((h - mu) ** 2, axis=1, keepdims=True)
    o_ref[...] = (h - mu) * jax.lax.rsqrt(var + 1e-6)


def _gat_layer(dst_state, src_state, cmat, wq, wk, f1, f2):
    nd, dd = dst_state.shape
    ns = src_state.shape[0]
    h = wk.shape[1]
    kmat = _matmul(src_state, wk)
    bd = _pick_bd(nd)
    ffn = f1.shape[1]
    return pl.pallas_call(
        _gat_body,
        grid=(nd // bd,),
        in_specs=[
            pl.BlockSpec((bd, dd), lambda i: (i, 0)),
            pl.BlockSpec((ns, h), lambda i: (0, 0)),
            pl.BlockSpec((bd, ns), lambda i: (i, 0)),
            pl.BlockSpec((dd, h), lambda i: (0, 0)),
            pl.BlockSpec((dd, ffn), lambda i: (0, 0)),
            pl.BlockSpec((ffn, dd), lambda i: (0, 0)),
        ],
        out_specs=pl.BlockSpec((bd, dd), lambda i: (i, 0)),
        out_shape=jax.ShapeDtypeStruct((nd, dd), jnp.float32),
    )(dst_state, kmat, cmat, wq, f1, f2)


def _build_counts(edst, esrc, nd, ns):
    return jnp.zeros((nd, ns), jnp.float32).at[edst, esrc].add(1.0)


def kernel(word_feat, sent_raw, edge_src, edge_dst, W_proj, Wq_ws, Wk_ws,
           F1_s, F2_s, Wq_sw, Wk_sw, F1_w, F2_w, W_head, b_head):
    nw = word_feat.shape[0]
    ns = sent_raw.shape[0]
    esrc = edge_src.astype(jnp.int32)
    edst = edge_dst.astype(jnp.int32)
    c_sw = _build_counts(edst, esrc, ns, nw)
    c_ws = _build_counts(esrc, edst, nw, ns)
    sent_feature = _matmul(sent_raw, W_proj)
    word_state = word_feat
    sent_state = _gat_layer(sent_feature, word_state, c_sw,
                            Wq_ws, Wk_ws, F1_s, F2_s)
    for _ in range(_N_ROUNDS):
        word_state = _gat_layer(word_state, sent_state, c_ws,
                                Wq_sw, Wk_sw, F1_w, F2_w)
        sent_state = _gat_layer(sent_state, word_state, c_sw,
                                Wq_ws, Wk_ws, F1_s, F2_s)
    return _matmul(sent_state, W_head) + b_head
